# Initial kernel scaffold; baseline (speedup 1.0000x reference)
import jax
import jax.numpy as jnp
from jax.experimental import pallas as pl


def kernel(nodes, neigh_idx, features_table, num_sample):
    raise NotImplementedError("write your pallas kernel here")


# speedup vs baseline: 4.7646x; 4.7646x over previous
import functools

import jax
import jax.numpy as jnp
from jax import lax
from jax.experimental import pallas as pl
from jax.experimental.pallas import tpu as pltpu
from jax.experimental.pallas import tpu_sc as plsc

NC = 2
NS = 16
NW = NC * NS
LANES = 16


@functools.partial(jax.jit, static_argnums=(2, 3, 4, 5))
def _mean_agg(flat_idx, table, b_per_w, chunk_rows, S, D):
    B_pad = NW * b_per_w
    n_chunks = b_per_w // chunk_rows
    mesh = plsc.VectorSubcoreMesh(
        core_axis_name="c", subcore_axis_name="s",
        num_cores=NC, num_subcores=NS)

    @functools.partial(
        pl.kernel,
        out_type=jax.ShapeDtypeStruct((B_pad, D), jnp.float32),
        mesh=mesh,
        scratch_types=[
            pltpu.VMEM((chunk_rows * S,), jnp.int32),
            pltpu.VMEM((chunk_rows * S, D), jnp.float32),
            pltpu.VMEM((chunk_rows, D), jnp.float32),
            pltpu.SemaphoreType.DMA,
        ],
    )
    def body(idx_hbm, table_hbm, out_hbm, idx_v, rows_v, acc_v, sem):
        wid = lax.axis_index("s") * NC + lax.axis_index("c")
        row0 = wid * b_per_w

        def chunk_step(i, carry):
            base_row = row0 + i * chunk_rows
            pltpu.sync_copy(idx_hbm.at[pl.ds(base_row * S, chunk_rows * S)],
                            idx_v)
            pltpu.async_copy(table_hbm.at[idx_v], rows_v, sem).wait()

            def row_step(r, c2):
                rS = r * S
                for j in range(D // LANES):
                    sl = pl.ds(j * LANES, LANES)
                    v = rows_v[rS, sl]
                    for s in range(1, S):
                        v = v + rows_v[rS + s, sl]
                    acc_v[r, sl] = v / jnp.float32(S)
                return c2

            lax.fori_loop(0, chunk_rows, row_step, 0)
            pltpu.sync_copy(acc_v, out_hbm.at[pl.ds(base_row, chunk_rows)])
            return carry

        lax.fori_loop(0, n_chunks, chunk_step, 0)

    return body(flat_idx, table)


def kernel(nodes, neigh_idx, features_table, num_sample):
    del nodes, num_sample
    B, S = neigh_idx.shape
    D = features_table.shape[1]
    chunk_rows = (128 // S) // 8 * 8 or 8
    per_w = -(-B // NW)
    b_per_w = -(-per_w // chunk_rows) * chunk_rows
    B_pad = NW * b_per_w
    flat = neigh_idx.astype(jnp.int32).reshape(-1)
    flat = jnp.pad(flat, (0, B_pad * S - flat.shape[0]))
    out = _mean_agg(flat, features_table, b_per_w, chunk_rows, S, D)
    return out[:B]


# speedup vs baseline: 8.9870x; 1.8862x over previous
import functools

import jax
import jax.numpy as jnp
from jax import lax
from jax.experimental import pallas as pl
from jax.experimental.pallas import tpu as pltpu
from jax.experimental.pallas import tpu_sc as plsc

NC = 2
NS = 16
NW = NC * NS
LANES = 16
NBUF = 4


@functools.partial(jax.jit, static_argnums=(2, 3, 4, 5))
def _mean_agg(flat_idx, table, b_per_w, C, S, D):
    B_pad = NW * b_per_w
    CS = C * S
    n_chunks = b_per_w // C
    assert n_chunks % NBUF == 0
    scale = jnp.float32(1.0 / S)
    mesh = plsc.VectorSubcoreMesh(
        core_axis_name="c", subcore_axis_name="s",
        num_cores=NC, num_subcores=NS)

    @functools.partial(
        pl.kernel,
        out_type=jax.ShapeDtypeStruct((B_pad, D), jnp.float32),
        mesh=mesh,
        scratch_types=[
            pltpu.VMEM((b_per_w * S,), jnp.int32),
            pltpu.VMEM((NBUF, CS, D), jnp.float32),
            pltpu.VMEM((NBUF, C, D), jnp.float32),
            [pltpu.SemaphoreType.DMA] * NBUF,
            [pltpu.SemaphoreType.DMA] * NBUF,
        ],
    )
    def body(idx_hbm, table_hbm, out_hbm, idx_all, rows, acc, gsems, ssems):
        wid = lax.axis_index("s") * NC + lax.axis_index("c")
        row0 = wid * b_per_w
        pltpu.sync_copy(idx_hbm.at[pl.ds(row0 * S, b_per_w * S)], idx_all)

        def gather_start(g, b):
            pltpu.async_copy(
                table_hbm.at[idx_all.at[pl.ds(g * CS, CS)]],
                rows.at[b], gsems[b])

        for b in range(NBUF):
            gather_start(b, b)

        def step(o, carry):
            for b in range(NBUF):
                g = o * NBUF + b
                pltpu.make_async_copy(
                    table_hbm.at[pl.ds(0, CS)], rows.at[b], gsems[b]).wait()
                @pl.when(g >= NBUF)
                def _():
                    pltpu.make_async_copy(
                        acc.at[b], out_hbm.at[pl.ds(row0, C)],
                        ssems[b]).wait()

                def row_step(r, c2):
                    rS = r * S
                    for j in range(D // LANES):
                        sl = pl.ds(j * LANES, LANES)
                        v = rows[b, rS, sl]
                        for s in range(1, S):
                            v = v + rows[b, rS + s, sl]
                        acc[b, r, sl] = v * scale
                    return c2

                lax.fori_loop(0, C, row_step, 0)
                pltpu.async_copy(
                    acc.at[b], out_hbm.at[pl.ds(row0 + g * C, C)], ssems[b])

                g2 = g + NBUF
                @pl.when(g2 < n_chunks)
                def _():
                    gather_start(g2, b)
            return carry

        lax.fori_loop(0, n_chunks // NBUF, step, 0)
        for b in range(NBUF):
            pltpu.make_async_copy(
                acc.at[b], out_hbm.at[pl.ds(row0, C)], ssems[b]).wait()

    return body(flat_idx, table)


def kernel(nodes, neigh_idx, features_table, num_sample):
    del nodes, num_sample
    B, S = neigh_idx.shape
    D = features_table.shape[1]
    C = (128 // S) // 8 * 8 or 8
    per_w = -(-B // NW)
    b_per_w = -(-per_w // (C * NBUF)) * (C * NBUF)
    B_pad = NW * b_per_w
    flat = neigh_idx.astype(jnp.int32).reshape(-1)
    flat = jnp.pad(flat, (0, B_pad * S - flat.shape[0]))
    out = _mean_agg(flat, features_table, b_per_w, C, S, D)
    return out[:B]
